# baseline (speedup 1.0000x reference)
import jax
import jax.numpy as jnp
from jax import lax
from jax.experimental import pallas as pl

B = 1024
P = 50
NE = 50
NR = 50
NUM_CMD = 7
NUM_TGT = 1000
TFD = 256
AD = 128
GATHER = 1
ATTACK = 2
BUILD_BUILDING = 3
BUILD_UNIT = 4
MOVE = 5

BB = 16
RB = BB * P
GRID = B // BB


def _body(pk_ref, ef_ref, rf_ref, cemb_ref, temb_ref, vxyt_ref, gxy_ref,
          bxy_ref, ve_ref, ge_ref, be_ref, vr_ref, gr_ref, br_ref, out_ref):
    f32 = jnp.float32
    bf16 = jnp.bfloat16
    i32 = jnp.int32
    pk = pk_ref[0]
    cmd = pk[0:1, :]
    tgt = pk[1:2, :]
    ia = pk[2:3, :]
    ig = pk[3:4, :]
    x = lax.bitcast_convert_type(pk[4:5, :], f32)
    y = lax.bitcast_convert_type(pk[5:6, :], f32)
    tt_b = (cmd == BUILD_BUILDING) | (cmd == BUILD_UNIT)
    xy_m = ((cmd == BUILD_BUILDING) | (cmd == MOVE)).astype(f32)
    e_b = cmd == ATTACK
    r_b = cmd == GATHER

    ioc = lax.broadcasted_iota(i32, (NUM_CMD, RB), 0)
    oh_c = (ioc == cmd).astype(bf16)
    o1 = lax.dot_general(oh_c, cemb_ref[...],
                         (((0,), (0,)), ((), ())), preferred_element_type=f32)
    out_ref[:, :, 0:AD] = o1.reshape(P, BB, AD)

    tgt_m = jnp.where(tt_b, tgt, -1)
    iot = lax.broadcasted_iota(i32, (NUM_TGT, RB), 0)
    oh_t = (iot == tgt_m).astype(bf16)
    o2 = lax.dot_general(oh_t, temb_ref[...],
                         (((0,), (0,)), ((), ())), preferred_element_type=f32)
    out_ref[:, :, AD:2 * AD] = o2.reshape(P, BB, AD)

    vxyt = vxyt_ref[...]
    sxy = gxy_ref[0, 0] / jnp.sqrt(jnp.sum(vxyt * vxyt))
    w3 = jnp.concatenate([vxyt * sxy, bxy_ref[...]], axis=0)
    d3 = jnp.concatenate([x * xy_m, y * xy_m, xy_m], axis=0)
    o3 = lax.dot_general(d3.astype(bf16), w3.astype(bf16),
                         (((0,), (0,)), ((), ())), preferred_element_type=f32)
    out_ref[:, :, 2 * AD:3 * AD] = o3.reshape(P, BB, AD)

    ve = ve_ref[...]
    se = ge_ref[0, 0] / jnp.sqrt(jnp.sum(ve * ve))
    vr = vr_ref[...]
    sr = gr_ref[0, 0] / jnp.sqrt(jnp.sum(vr * vr))
    ef2 = ef_ref[...].reshape(RB, TFD).astype(bf16)
    rf2 = rf_ref[...].reshape(RB, TFD).astype(bf16)
    efw = lax.dot_general(ef2, ve.astype(bf16), (((1,), (1,)), ((), ())),
                          preferred_element_type=f32) * se
    rfw = lax.dot_general(rf2, vr.astype(bf16), (((1,), (1,)), ((), ())),
                          preferred_element_type=f32) * sr
    bcol = lax.broadcasted_iota(i32, (1, RB), 1) & (BB - 1)
    gie = jnp.where(e_b, ia * BB + bcol, -1)
    gir = jnp.where(r_b, ig * BB + bcol, -1)
    ior = lax.broadcasted_iota(i32, (RB, RB), 0)
    oh_e = (ior == gie).astype(bf16)
    oh_r = (ior == gir).astype(bf16)
    d4 = jnp.concatenate([e_b.astype(bf16), r_b.astype(bf16)], axis=0)
    w4 = jnp.concatenate([be_ref[...], br_ref[...]], axis=0)
    o4 = (lax.dot_general(oh_e, efw.astype(bf16),
                          (((0,), (0,)), ((), ())), preferred_element_type=f32)
          + lax.dot_general(oh_r, rfw.astype(bf16),
                            (((0,), (0,)), ((), ())),
                            preferred_element_type=f32)
          + lax.dot_general(d4, w4.astype(bf16),
                            (((0,), (0,)), ((), ())),
                            preferred_element_type=f32))
    out_ref[:, :, 3 * AD:4 * AD] = o4.reshape(P, BB, AD)


@jax.jit
def _run(pk, enemy_feat, resource_feat,
         cmd_emb, tgt_emb, v_xyt, g_xy, b_xy, v_e, g_e, b_e, v_r, g_r, b_r):
    full = lambda a, b: pl.BlockSpec((a, b), lambda i: (0, 0))
    return pl.pallas_call(
        _body,
        grid=(GRID,),
        in_specs=[
            pl.BlockSpec((1, 6, RB), lambda i: (i, 0, 0)),
            pl.BlockSpec((P, BB, TFD), lambda i: (0, i, 0)),
            pl.BlockSpec((P, BB, TFD), lambda i: (0, i, 0)),
            full(NUM_CMD, AD),
            full(NUM_TGT, AD),
            full(2, AD),
            full(1, 1),
            full(1, AD),
            full(AD, TFD),
            full(1, 1),
            full(1, AD),
            full(AD, TFD),
            full(1, 1),
            full(1, AD),
        ],
        out_specs=pl.BlockSpec((P, BB, 4 * AD), lambda i: (0, i, 0)),
        out_shape=jax.ShapeDtypeStruct((P, B, 4 * AD), jnp.float32),
    )(pk, enemy_feat, resource_feat,
      cmd_emb, tgt_emb, v_xyt, g_xy, b_xy, v_e, g_e, b_e, v_r, g_r, b_r)


def kernel(num_real_unit, cmd_type, target_type, x, y, target_attack_idx,
           target_gather_idx, enemy_feat, resource_feat,
           cmd_emb, tgt_emb, v_xy, g_xy, b_xy, v_e, g_e, b_e, v_r, g_r, b_r):
    del num_real_unit
    i32 = jnp.int32
    bc = lambda a: lax.bitcast_convert_type(a, i32)
    pk6 = jnp.stack([cmd_type.astype(i32), target_type.astype(i32),
                     target_attack_idx.astype(i32),
                     target_gather_idx.astype(i32), bc(x), bc(y)], axis=0)
    pk = (pk6.transpose(0, 2, 1)
          .reshape(6, P, GRID, BB)
          .transpose(2, 0, 1, 3)
          .reshape(GRID, 6, RB))
    outT = _run(pk,
                enemy_feat.transpose(1, 0, 2),
                resource_feat.transpose(1, 0, 2),
                cmd_emb.astype(jnp.bfloat16), tgt_emb.astype(jnp.bfloat16),
                v_xy.T,
                g_xy.reshape(1, 1), b_xy.reshape(1, AD),
                v_e, g_e.reshape(1, 1), b_e.reshape(1, AD),
                v_r, g_r.reshape(1, 1), b_r.reshape(1, AD))
    return outT.transpose(1, 0, 2)


# speedup vs baseline: 1.0087x; 1.0087x over previous
import jax
import jax.numpy as jnp
from jax import lax
from jax.experimental import pallas as pl

B = 1024
P = 50
NE = 50
NR = 50
NUM_CMD = 7
NUM_TGT = 1000
TFD = 256
AD = 128
GATHER = 1
ATTACK = 2
BUILD_BUILDING = 3
BUILD_UNIT = 4
MOVE = 5

BB = 16
RB = BB * P
GRID = B // BB


def _body(pk_ref, ef_ref, rf_ref, cemb_ref, temb_ref, vxyt_ref, gxy_ref,
          bxy_ref, ve_ref, ge_ref, be_ref, vr_ref, gr_ref, br_ref, out_ref):
    f32 = jnp.float32
    bf16 = jnp.bfloat16
    i32 = jnp.int32
    pk = pk_ref[0]
    cmd = pk[0:1, :]
    tgt = pk[1:2, :]
    ia = pk[2:3, :]
    ig = pk[3:4, :]
    x = lax.bitcast_convert_type(pk[4:5, :], f32)
    y = lax.bitcast_convert_type(pk[5:6, :], f32)
    tt_m = ((cmd == BUILD_BUILDING) | (cmd == BUILD_UNIT)).astype(bf16)
    xy_m = ((cmd == BUILD_BUILDING) | (cmd == MOVE)).astype(f32)
    e_m = (cmd == ATTACK).astype(bf16)
    r_m = (cmd == GATHER).astype(bf16)

    ioc = lax.broadcasted_iota(i32, (NUM_CMD, RB), 0)
    oh_c = (ioc == cmd).astype(bf16)
    o1 = lax.dot_general(oh_c, cemb_ref[...].astype(bf16),
                         (((0,), (0,)), ((), ())), preferred_element_type=f32)
    out_ref[:, :, 0:AD] = o1.reshape(P, BB, AD)

    iot = lax.broadcasted_iota(i32, (NUM_TGT, RB), 0)
    oh_t = (iot == tgt).astype(bf16) * tt_m
    o2 = lax.dot_general(oh_t, temb_ref[...].astype(bf16),
                         (((0,), (0,)), ((), ())), preferred_element_type=f32)
    out_ref[:, :, AD:2 * AD] = o2.reshape(P, BB, AD)

    vxyt = vxyt_ref[...]
    sxy = gxy_ref[0, 0] / jnp.sqrt(jnp.sum(vxyt * vxyt))
    w3 = jnp.concatenate([vxyt * sxy, bxy_ref[...]], axis=0)
    d3 = jnp.concatenate([x * xy_m, y * xy_m, xy_m], axis=0)
    o3 = lax.dot_general(d3.astype(bf16), w3.astype(bf16),
                         (((0,), (0,)), ((), ())), preferred_element_type=f32)
    out_ref[:, :, 2 * AD:3 * AD] = o3.reshape(P, BB, AD)

    ve = ve_ref[...]
    se = ge_ref[0, 0] / jnp.sqrt(jnp.sum(ve * ve))
    vr = vr_ref[...]
    sr = gr_ref[0, 0] / jnp.sqrt(jnp.sum(vr * vr))
    ef2 = ef_ref[...].reshape(RB, TFD).astype(bf16)
    rf2 = rf_ref[...].reshape(RB, TFD).astype(bf16)
    efw = lax.dot_general(ef2, ve.astype(bf16), (((1,), (1,)), ((), ())),
                          preferred_element_type=f32) * se
    rfw = lax.dot_general(rf2, vr.astype(bf16), (((1,), (1,)), ((), ())),
                          preferred_element_type=f32) * sr
    bcol = lax.broadcasted_iota(i32, (1, RB), 1) & (BB - 1)
    gie = ia * BB + bcol
    gir = ig * BB + bcol
    ior = lax.broadcasted_iota(i32, (RB, RB), 0)
    oh_e = (ior == gie).astype(bf16) * e_m
    oh_r = (ior == gir).astype(bf16) * r_m
    oh_e_aug = jnp.concatenate([oh_e, e_m], axis=0)
    oh_r_aug = jnp.concatenate([oh_r, r_m], axis=0)
    efw_aug = jnp.concatenate([efw, be_ref[...]], axis=0)
    rfw_aug = jnp.concatenate([rfw, br_ref[...]], axis=0)
    o4 = (lax.dot_general(oh_e_aug, efw_aug.astype(bf16),
                          (((0,), (0,)), ((), ())), preferred_element_type=f32)
          + lax.dot_general(oh_r_aug, rfw_aug.astype(bf16),
                            (((0,), (0,)), ((), ())),
                            preferred_element_type=f32))
    out_ref[:, :, 3 * AD:4 * AD] = o4.reshape(P, BB, AD)


@jax.jit
def _run(pk, enemy_feat, resource_feat,
         cmd_emb, tgt_emb, v_xyt, g_xy, b_xy, v_e, g_e, b_e, v_r, g_r, b_r):
    full = lambda a, b: pl.BlockSpec((a, b), lambda i: (0, 0))
    return pl.pallas_call(
        _body,
        grid=(GRID,),
        in_specs=[
            pl.BlockSpec((1, 6, RB), lambda i: (i, 0, 0)),
            pl.BlockSpec((P, BB, TFD), lambda i: (0, i, 0)),
            pl.BlockSpec((P, BB, TFD), lambda i: (0, i, 0)),
            full(NUM_CMD, AD),
            full(NUM_TGT, AD),
            full(2, AD),
            full(1, 1),
            full(1, AD),
            full(AD, TFD),
            full(1, 1),
            full(1, AD),
            full(AD, TFD),
            full(1, 1),
            full(1, AD),
        ],
        out_specs=pl.BlockSpec((P, BB, 4 * AD), lambda i: (0, i, 0)),
        out_shape=jax.ShapeDtypeStruct((P, B, 4 * AD), jnp.float32),
    )(pk, enemy_feat, resource_feat,
      cmd_emb, tgt_emb, v_xyt, g_xy, b_xy, v_e, g_e, b_e, v_r, g_r, b_r)


def kernel(num_real_unit, cmd_type, target_type, x, y, target_attack_idx,
           target_gather_idx, enemy_feat, resource_feat,
           cmd_emb, tgt_emb, v_xy, g_xy, b_xy, v_e, g_e, b_e, v_r, g_r, b_r):
    del num_real_unit
    i32 = jnp.int32
    bc = lambda a: lax.bitcast_convert_type(a, i32)
    pk6 = jnp.stack([cmd_type.astype(i32), target_type.astype(i32),
                     target_attack_idx.astype(i32),
                     target_gather_idx.astype(i32), bc(x), bc(y)], axis=0)
    pk = (pk6.transpose(0, 2, 1)
          .reshape(6, P, GRID, BB)
          .transpose(2, 0, 1, 3)
          .reshape(GRID, 6, RB))
    outT = _run(pk,
                enemy_feat.transpose(1, 0, 2),
                resource_feat.transpose(1, 0, 2),
                cmd_emb, tgt_emb, v_xy.T,
                g_xy.reshape(1, 1), b_xy.reshape(1, AD),
                v_e, g_e.reshape(1, 1), b_e.reshape(1, AD),
                v_r, g_r.reshape(1, 1), b_r.reshape(1, AD))
    return outT.transpose(1, 0, 2)
